# baseline (speedup 1.0000x reference)
import jax
import jax.numpy as jnp
from jax import lax
from jax.experimental import pallas as pl
from jax.experimental.pallas import tpu as pltpu
from jax.experimental.pallas import tpu_sc as plsc

B, N, D, P = 4, 16384, 64, 64
C = B * P
L = 16
V = N // L
RADIX = 256
CNT = RADIX * L
PART = 3
PS = 344
MAIN = V - (PART - 1) * PS
NC, NS = 2, 16
NW = NC * NS
CPW = C // NW
NT = 2048

_SIGN = -2**31


def _normalize_theta(th):
    norm = jnp.sqrt(jnp.sum(th * th, axis=1, keepdims=True))
    return th / jnp.maximum(norm, 1e-8)


def _proj_body(x_ref, y_ref, th_ref, xo_ref, yo_ref):
    th = _normalize_theta(th_ref[...])
    dn = (((1,), (1,)), ((), ()))
    xo_ref[0] = lax.dot_general(th, x_ref[0], dn,
                                preferred_element_type=jnp.float32,
                                precision=lax.Precision.HIGHEST)
    yo_ref[0] = lax.dot_general(th, y_ref[0], dn,
                                preferred_element_type=jnp.float32,
                                precision=lax.Precision.HIGHEST)


def _project(x, y, theta_raw):
    grid = (B, N // NT)
    return pl.pallas_call(
        _proj_body,
        grid=grid,
        in_specs=[
            pl.BlockSpec((1, NT, D), lambda b, n: (b, n, 0)),
            pl.BlockSpec((1, NT, D), lambda b, n: (b, n, 0)),
            pl.BlockSpec((P, D), lambda b, n: (0, 0)),
        ],
        out_specs=[
            pl.BlockSpec((1, P, NT), lambda b, n: (b, 0, n)),
            pl.BlockSpec((1, P, NT), lambda b, n: (b, 0, n)),
        ],
        out_shape=[
            jax.ShapeDtypeStruct((B, P, N), jnp.float32),
            jax.ShapeDtypeStruct((B, P, N), jnp.float32),
        ],
    )(x, y, theta_raw)


def _encode(u):
    m = lax.shift_right_arithmetic(u, 31)
    return u ^ (m | _SIGN)


def _decode(e):
    m = lax.shift_right_arithmetic(e, 31)
    return e ^ (~m | _SIGN)


def _sc_body(x_hbm, y_hbm, out_hbm, ak, av, bk, bv, yb, ck, cntx, cnty):
    wid = lax.axis_index("s") * NC + lax.axis_index("c")
    lane = lax.iota(jnp.int32, L)
    ones = jnp.ones((L,), jnp.int32)
    zeros = jnp.zeros((L,), jnp.int32)

    def rank_to_mem(r):
        return (r & (V - 1)) * L + lax.shift_right_logical(r, 10)

    def radix_pass(xk_src, xv_src, xv_dst, yk_src, shift, first, last,
                   xk_dst=None, yk_dst=None, out=None):
        def loadx(v):
            k = xk_src[pl.ds(v * L, L)]
            return _encode(k) if first else k

        def loady(v):
            k = yk_src[pl.ds(v * L, L)]
            return _encode(k) if first else k

        @plsc.parallel_loop(0, PART * CNT // L, unroll=4)
        def _zero(i):
            sl = pl.ds(i * L, L)
            cntx[sl] = zeros
            cnty[sl] = zeros

        def hist_one(v, p):
            kx = loadx(v)
            dx = lax.shift_right_logical(kx, shift) & 255
            plsc.addupdate_scatter(cntx, [p * CNT + dx * L + lane], ones)
            ky = loady(v)
            dy = lax.shift_right_logical(ky, shift) & 255
            plsc.addupdate_scatter(cnty, [p * CNT + dy * L + lane], ones)

        @plsc.parallel_loop(0, MAIN, unroll=2)
        def _hist(i):
            for p in range(PART):
                hist_one(i + p * PS, p)

        @plsc.parallel_loop(MAIN, PS, unroll=2)
        def _hist_epi(i):
            for p in range(PART - 1):
                hist_one(i + p * PS, p)

        @plsc.parallel_loop(0, CNT // L, carry=(jnp.int32(0), jnp.int32(0)))
        def _scan(i, carry):
            cax, cay = carry
            sls = [pl.ds(p * CNT + i * L, L) for p in range(PART)]
            cxs = [cntx[sl] for sl in sls]
            sx = sum(cxs)
            px = plsc.cumsum(sx) - sx + cax
            for p in range(PART):
                cntx[sls[p]] = px
                px = px + cxs[p]
            cax = cax + jnp.sum(sx)
            cys = [cnty[sl] for sl in sls]
            sy = sum(cys)
            py = plsc.cumsum(sy) - sy + cay
            for p in range(PART):
                cnty[sls[p]] = py
                py = py + cys[p]
            cay = cay + jnp.sum(sy)
            return cax, cay

        def permx(v, p, store_keys):
            kx = loadx(v)
            dx = lax.shift_right_logical(kx, shift) & 255
            ix = p * CNT + dx * L + lane
            ox = plsc.load_gather(cntx, [ix])
            plsc.store_scatter(cntx, [ix], ox + 1)
            memx = rank_to_mem(ox)
            if store_keys:
                plsc.store_scatter(xk_dst, [memx], kx)
            val = (v * L + lane) if first else xv_src[pl.ds(v * L, L)]
            plsc.store_scatter(xv_dst, [memx], val)

        def permy(v, p):
            ky = loady(v)
            dy = lax.shift_right_logical(ky, shift) & 255
            iy = p * CNT + dy * L + lane
            oy = plsc.load_gather(cnty, [iy])
            plsc.store_scatter(cnty, [iy], oy + 1)
            memy = rank_to_mem(oy)
            if not last:
                plsc.store_scatter(yk_dst, [memy], ky)
            else:
                pos = plsc.load_gather(xv_dst, [memy])
                plsc.store_scatter(out, [pos], _decode(ky))

        if not last:
            def perm(i, _):
                for p in range(PART):
                    v = i + p * PS
                    permx(v, p, True)
                    permy(v, p)
                return 0
            lax.fori_loop(0, MAIN, perm, 0)

            def perm_epi(i, _):
                for p in range(PART - 1):
                    v = i + p * PS
                    permx(v, p, True)
                    permy(v, p)
                return 0
            lax.fori_loop(MAIN, PS, perm_epi, 0)
        else:
            def perm_x(i, _):
                for p in range(PART):
                    permx(i + p * PS, p, False)
                return 0
            lax.fori_loop(0, MAIN, perm_x, 0)

            def perm_x_epi(i, _):
                for p in range(PART - 1):
                    permx(i + p * PS, p, False)
                return 0
            lax.fori_loop(MAIN, PS, perm_x_epi, 0)

            def perm_y(i, _):
                for p in range(PART):
                    permy(i + p * PS, p)
                return 0
            lax.fori_loop(0, MAIN, perm_y, 0)

            def perm_y_epi(i, _):
                for p in range(PART - 1):
                    permy(i + p * PS, p)
                return 0
            lax.fori_loop(MAIN, PS, perm_y_epi, 0)

    def column(j, _):
        col = wid * CPW + j
        pltpu.sync_copy(x_hbm.at[col], ak)
        pltpu.sync_copy(y_hbm.at[col], yb)
        radix_pass(ak, None, bv, yb, 0, True, False, xk_dst=bk, yk_dst=ck)
        radix_pass(bk, bv, av, ck, 8, False, False, xk_dst=ak, yk_dst=yb)
        radix_pass(ak, av, bv, yb, 16, False, False, xk_dst=bk, yk_dst=ck)
        radix_pass(bk, bv, av, ck, 24, False, True, out=ak)
        pltpu.sync_copy(ak, out_hbm.at[col])
        return 0

    lax.fori_loop(0, CPW, column, 0)


def _sc_transport(xT_bits, yT_bits):
    mesh = plsc.VectorSubcoreMesh(core_axis_name="c", subcore_axis_name="s",
                                  num_cores=NC, num_subcores=NS)
    f = pl.kernel(
        _sc_body,
        out_type=jax.ShapeDtypeStruct((C, N), jnp.int32),
        mesh=mesh,
        compiler_params=pltpu.CompilerParams(needs_layout_passes=False),
        scratch_types=[
            pltpu.VMEM((N,), jnp.int32),
            pltpu.VMEM((N,), jnp.int32),
            pltpu.VMEM((N,), jnp.int32),
            pltpu.VMEM((N,), jnp.int32),
            pltpu.VMEM((N,), jnp.int32),
            pltpu.VMEM((N,), jnp.int32),
            pltpu.VMEM((PART * CNT,), jnp.int32),
            pltpu.VMEM((PART * CNT,), jnp.int32),
        ],
    )
    return f(xT_bits, yT_bits)


def _assemble_body(scale_ref, t_ref, xp_ref, x_ref, th_ref, o_ref):
    th = _normalize_theta(th_ref[...])
    transported = lax.bitcast_convert_type(t_ref[0], jnp.float32)
    diff = transported - xp_ref[0]
    dn = (((0,), (0,)), ((), ()))
    t = lax.dot_general(diff, th, dn,
                        preferred_element_type=jnp.float32,
                        precision=lax.Precision.HIGHEST)
    o_ref[0] = x_ref[0] + t * scale_ref[0]


def _assemble(transT_bits, xT, x, theta_raw, n_projections):
    grid = (B, N // NT)
    scale = (1.0 / jnp.asarray(n_projections, jnp.float32)).reshape(1)
    return pl.pallas_call(
        _assemble_body,
        grid=grid,
        in_specs=[
            pl.BlockSpec(memory_space=pltpu.SMEM),
            pl.BlockSpec((1, P, NT), lambda b, n: (b, 0, n)),
            pl.BlockSpec((1, P, NT), lambda b, n: (b, 0, n)),
            pl.BlockSpec((1, NT, D), lambda b, n: (b, n, 0)),
            pl.BlockSpec((P, D), lambda b, n: (0, 0)),
        ],
        out_specs=pl.BlockSpec((1, NT, D), lambda b, n: (b, n, 0)),
        out_shape=jax.ShapeDtypeStruct((B, N, D), jnp.float32),
    )(scale, transT_bits, xT, x, theta_raw)


def kernel(x_batch, y_batch, eps, n_projections, theta_raw):
    del eps
    xT, yT = _project(x_batch, y_batch, theta_raw)
    xT_bits = lax.bitcast_convert_type(xT, jnp.int32).reshape(C, N)
    yT_bits = lax.bitcast_convert_type(yT, jnp.int32).reshape(C, N)
    transT_bits = _sc_transport(xT_bits, yT_bits).reshape(B, P, N)
    return _assemble(transT_bits, xT, x_batch, theta_raw, n_projections)


# speedup vs baseline: 1.0095x; 1.0095x over previous
import jax
import jax.numpy as jnp
from jax import lax
from jax.experimental import pallas as pl
from jax.experimental.pallas import tpu as pltpu
from jax.experimental.pallas import tpu_sc as plsc

B, N, D, P = 4, 16384, 64, 64
C = B * P
L = 16
V = N // L
RADIX = 256
CNT = RADIX * L
PART = 2
H = V // PART
NC, NS = 2, 16
NW = NC * NS
CPW = C // NW
NT = 2048

_SIGN = -2**31


def _normalize_theta(th):
    norm = jnp.sqrt(jnp.sum(th * th, axis=1, keepdims=True))
    return th / jnp.maximum(norm, 1e-8)


def _proj_body(x_ref, y_ref, th_ref, xo_ref, yo_ref):
    th = _normalize_theta(th_ref[...])
    dn = (((1,), (1,)), ((), ()))
    xo_ref[0] = lax.dot_general(th, x_ref[0], dn,
                                preferred_element_type=jnp.float32,
                                precision=lax.Precision.HIGHEST)
    yo_ref[0] = lax.dot_general(th, y_ref[0], dn,
                                preferred_element_type=jnp.float32,
                                precision=lax.Precision.HIGHEST)


def _project(x, y, theta_raw):
    grid = (B, N // NT)
    return pl.pallas_call(
        _proj_body,
        grid=grid,
        in_specs=[
            pl.BlockSpec((1, NT, D), lambda b, n: (b, n, 0)),
            pl.BlockSpec((1, NT, D), lambda b, n: (b, n, 0)),
            pl.BlockSpec((P, D), lambda b, n: (0, 0)),
        ],
        out_specs=[
            pl.BlockSpec((1, P, NT), lambda b, n: (b, 0, n)),
            pl.BlockSpec((1, P, NT), lambda b, n: (b, 0, n)),
        ],
        out_shape=[
            jax.ShapeDtypeStruct((B, P, N), jnp.float32),
            jax.ShapeDtypeStruct((B, P, N), jnp.float32),
        ],
    )(x, y, theta_raw)


def _encode(u):
    m = lax.shift_right_arithmetic(u, 31)
    return u ^ (m | _SIGN)


def _decode(e):
    m = lax.shift_right_arithmetic(e, 31)
    return e ^ (~m | _SIGN)


def _sc_body(x_hbm, y_hbm, out_hbm, ak, av, bk, bv, yb, ck, cntx, cnty):
    wid = lax.axis_index("s") * NC + lax.axis_index("c")
    lane = lax.iota(jnp.int32, L)
    ones = jnp.ones((L,), jnp.int32)
    zeros = jnp.zeros((L,), jnp.int32)

    def rank_to_mem(r):
        return (r & (V - 1)) * L + lax.shift_right_logical(r, 10)

    def radix_pass(xk_src, xv_src, xv_dst, yk_src, shift, first, last,
                   xk_dst=None, yk_dst=None, out=None):
        def loadx(v):
            k = xk_src[pl.ds(v * L, L)]
            return _encode(k) if first else k

        def loady(v):
            k = yk_src[pl.ds(v * L, L)]
            return _encode(k) if first else k

        @plsc.parallel_loop(0, PART * CNT // L, unroll=4)
        def _zero(i):
            sl = pl.ds(i * L, L)
            cntx[sl] = zeros
            cnty[sl] = zeros

        def hist_one(v, p):
            kx = loadx(v)
            dx = lax.shift_right_logical(kx, shift) & 255
            plsc.addupdate_scatter(cntx, [p * CNT + dx * L + lane], ones)
            ky = loady(v)
            dy = lax.shift_right_logical(ky, shift) & 255
            plsc.addupdate_scatter(cnty, [p * CNT + dy * L + lane], ones)

        @plsc.parallel_loop(0, H, unroll=4)
        def _hist(i):
            for p in range(PART):
                hist_one(i + p * H, p)

        @plsc.parallel_loop(0, CNT // L, carry=(jnp.int32(0), jnp.int32(0)))
        def _scan(i, carry):
            cax, cay = carry
            sls = [pl.ds(p * CNT + i * L, L) for p in range(PART)]
            cxs = [cntx[sl] for sl in sls]
            sx = sum(cxs)
            px = plsc.cumsum(sx) - sx + cax
            for p in range(PART):
                cntx[sls[p]] = px
                px = px + cxs[p]
            cax = cax + jnp.sum(sx)
            cys = [cnty[sl] for sl in sls]
            sy = sum(cys)
            py = plsc.cumsum(sy) - sy + cay
            for p in range(PART):
                cnty[sls[p]] = py
                py = py + cys[p]
            cay = cay + jnp.sum(sy)
            return cax, cay

        def permx(v, p, store_keys):
            kx = loadx(v)
            dx = lax.shift_right_logical(kx, shift) & 255
            ix = p * CNT + dx * L + lane
            ox = plsc.load_gather(cntx, [ix])
            plsc.store_scatter(cntx, [ix], ox + 1)
            memx = rank_to_mem(ox)
            if store_keys:
                plsc.store_scatter(xk_dst, [memx], kx)
            val = (v * L + lane) if first else xv_src[pl.ds(v * L, L)]
            plsc.store_scatter(xv_dst, [memx], val)

        def permy(v, p):
            ky = loady(v)
            dy = lax.shift_right_logical(ky, shift) & 255
            iy = p * CNT + dy * L + lane
            oy = plsc.load_gather(cnty, [iy])
            plsc.store_scatter(cnty, [iy], oy + 1)
            memy = rank_to_mem(oy)
            if not last:
                plsc.store_scatter(yk_dst, [memy], ky)
            else:
                pos = plsc.load_gather(xv_dst, [memy])
                plsc.store_scatter(out, [pos], _decode(ky))

        if not last:
            def perm(i, _):
                for u in range(2):
                    for p in range(PART):
                        v = i * 2 + u + p * H
                        permx(v, p, True)
                        permy(v, p)
                return 0
            lax.fori_loop(0, H // 2, perm, 0)
        else:
            def perm_x(i, _):
                for u in range(2):
                    for p in range(PART):
                        permx(i * 2 + u + p * H, p, False)
                return 0
            lax.fori_loop(0, H // 2, perm_x, 0)

            def perm_y(i, _):
                for u in range(2):
                    for p in range(PART):
                        permy(i * 2 + u + p * H, p)
                return 0
            lax.fori_loop(0, H // 2, perm_y, 0)

    def column(j, _):
        col = wid * CPW + j
        pltpu.sync_copy(x_hbm.at[col], ak)
        pltpu.sync_copy(y_hbm.at[col], yb)
        radix_pass(ak, None, bv, yb, 0, True, False, xk_dst=bk, yk_dst=ck)
        radix_pass(bk, bv, av, ck, 8, False, False, xk_dst=ak, yk_dst=yb)
        radix_pass(ak, av, bv, yb, 16, False, False, xk_dst=bk, yk_dst=ck)
        radix_pass(bk, bv, av, ck, 24, False, True, out=ak)
        pltpu.sync_copy(ak, out_hbm.at[col])
        return 0

    lax.fori_loop(0, CPW, column, 0)


def _sc_transport(xT_bits, yT_bits):
    mesh = plsc.VectorSubcoreMesh(core_axis_name="c", subcore_axis_name="s",
                                  num_cores=NC, num_subcores=NS)
    f = pl.kernel(
        _sc_body,
        out_type=jax.ShapeDtypeStruct((C, N), jnp.int32),
        mesh=mesh,
        compiler_params=pltpu.CompilerParams(needs_layout_passes=False),
        scratch_types=[
            pltpu.VMEM((N,), jnp.int32),
            pltpu.VMEM((N,), jnp.int32),
            pltpu.VMEM((N,), jnp.int32),
            pltpu.VMEM((N,), jnp.int32),
            pltpu.VMEM((N,), jnp.int32),
            pltpu.VMEM((N,), jnp.int32),
            pltpu.VMEM((PART * CNT,), jnp.int32),
            pltpu.VMEM((PART * CNT,), jnp.int32),
        ],
    )
    return f(xT_bits, yT_bits)


def _assemble_body(scale_ref, t_ref, xp_ref, x_ref, th_ref, o_ref):
    th = _normalize_theta(th_ref[...])
    transported = lax.bitcast_convert_type(t_ref[0], jnp.float32)
    diff = transported - xp_ref[0]
    dn = (((0,), (0,)), ((), ()))
    t = lax.dot_general(diff, th, dn,
                        preferred_element_type=jnp.float32,
                        precision=lax.Precision.HIGHEST)
    o_ref[0] = x_ref[0] + t * scale_ref[0]


def _assemble(transT_bits, xT, x, theta_raw, n_projections):
    grid = (B, N // NT)
    scale = (1.0 / jnp.asarray(n_projections, jnp.float32)).reshape(1)
    return pl.pallas_call(
        _assemble_body,
        grid=grid,
        in_specs=[
            pl.BlockSpec(memory_space=pltpu.SMEM),
            pl.BlockSpec((1, P, NT), lambda b, n: (b, 0, n)),
            pl.BlockSpec((1, P, NT), lambda b, n: (b, 0, n)),
            pl.BlockSpec((1, NT, D), lambda b, n: (b, n, 0)),
            pl.BlockSpec((P, D), lambda b, n: (0, 0)),
        ],
        out_specs=pl.BlockSpec((1, NT, D), lambda b, n: (b, n, 0)),
        out_shape=jax.ShapeDtypeStruct((B, N, D), jnp.float32),
    )(scale, transT_bits, xT, x, theta_raw)


def kernel(x_batch, y_batch, eps, n_projections, theta_raw):
    del eps
    xT, yT = _project(x_batch, y_batch, theta_raw)
    xT_bits = lax.bitcast_convert_type(xT, jnp.int32).reshape(C, N)
    yT_bits = lax.bitcast_convert_type(yT, jnp.int32).reshape(C, N)
    transT_bits = _sc_transport(xT_bits, yT_bits).reshape(B, P, N)
    return _assemble(transT_bits, xT, x_batch, theta_raw, n_projections)


# speedup vs baseline: 1.0190x; 1.0094x over previous
import jax
import jax.numpy as jnp
from jax import lax
from jax.experimental import pallas as pl
from jax.experimental.pallas import tpu as pltpu
from jax.experimental.pallas import tpu_sc as plsc

B, N, D, P = 4, 16384, 64, 64
C = B * P
L = 16
V = N // L
RADIX = 256
CNT = RADIX * L
PART = 2
H = V // PART
NC, NS = 2, 16
NW = NC * NS
CPW = C // NW
NT = 4096

_SIGN = -2**31


def _normalize_theta(th):
    norm = jnp.sqrt(jnp.sum(th * th, axis=1, keepdims=True))
    return th / jnp.maximum(norm, 1e-8)


def _proj_body(x_ref, y_ref, th_ref, xo_ref, yo_ref):
    th = _normalize_theta(th_ref[...])
    dn = (((1,), (1,)), ((), ()))
    xo_ref[0] = lax.dot_general(th, x_ref[0], dn,
                                preferred_element_type=jnp.float32,
                                precision=lax.Precision.HIGHEST)
    yo_ref[0] = lax.dot_general(th, y_ref[0], dn,
                                preferred_element_type=jnp.float32,
                                precision=lax.Precision.HIGHEST)


def _project(x, y, theta_raw):
    grid = (B, N // NT)
    return pl.pallas_call(
        _proj_body,
        grid=grid,
        in_specs=[
            pl.BlockSpec((1, NT, D), lambda b, n: (b, n, 0)),
            pl.BlockSpec((1, NT, D), lambda b, n: (b, n, 0)),
            pl.BlockSpec((P, D), lambda b, n: (0, 0)),
        ],
        out_specs=[
            pl.BlockSpec((1, P, NT), lambda b, n: (b, 0, n)),
            pl.BlockSpec((1, P, NT), lambda b, n: (b, 0, n)),
        ],
        out_shape=[
            jax.ShapeDtypeStruct((B, P, N), jnp.float32),
            jax.ShapeDtypeStruct((B, P, N), jnp.float32),
        ],
    )(x, y, theta_raw)


def _encode(u):
    m = lax.shift_right_arithmetic(u, 31)
    return u ^ (m | _SIGN)


def _decode(e):
    m = lax.shift_right_arithmetic(e, 31)
    return e ^ (~m | _SIGN)


def _sc_body(x_hbm, y_hbm, out_hbm, ak, av, bk, bv, yb, ck, cntx, cnty):
    wid = lax.axis_index("s") * NC + lax.axis_index("c")
    lane = lax.iota(jnp.int32, L)
    ones = jnp.ones((L,), jnp.int32)
    zeros = jnp.zeros((L,), jnp.int32)

    def rank_to_mem(r):
        return (r & (V - 1)) * L + lax.shift_right_logical(r, 10)

    def radix_pass(xk_src, xv_src, xv_dst, yk_src, shift, first, last,
                   xk_dst=None, yk_dst=None, out=None):
        def loadx(v):
            k = xk_src[pl.ds(v * L, L)]
            return _encode(k) if first else k

        def loady(v):
            k = yk_src[pl.ds(v * L, L)]
            return _encode(k) if first else k

        @plsc.parallel_loop(0, PART * CNT // L, unroll=4)
        def _zero(i):
            sl = pl.ds(i * L, L)
            cntx[sl] = zeros
            cnty[sl] = zeros

        def hist_one(v, p):
            kx = loadx(v)
            dx = lax.shift_right_logical(kx, shift) & 255
            plsc.addupdate_scatter(cntx, [p * CNT + dx * L + lane], ones)
            ky = loady(v)
            dy = lax.shift_right_logical(ky, shift) & 255
            plsc.addupdate_scatter(cnty, [p * CNT + dy * L + lane], ones)

        @plsc.parallel_loop(0, H, unroll=4)
        def _hist(i):
            for p in range(PART):
                hist_one(i + p * H, p)

        @plsc.parallel_loop(0, CNT // L, carry=(jnp.int32(0), jnp.int32(0)))
        def _scan(i, carry):
            cax, cay = carry
            sls = [pl.ds(p * CNT + i * L, L) for p in range(PART)]
            cxs = [cntx[sl] for sl in sls]
            sx = sum(cxs)
            px = plsc.cumsum(sx) - sx + cax
            for p in range(PART):
                cntx[sls[p]] = px
                px = px + cxs[p]
            cax = cax + jnp.sum(sx)
            cys = [cnty[sl] for sl in sls]
            sy = sum(cys)
            py = plsc.cumsum(sy) - sy + cay
            for p in range(PART):
                cnty[sls[p]] = py
                py = py + cys[p]
            cay = cay + jnp.sum(sy)
            return cax, cay

        def permx(v, p, store_keys):
            kx = loadx(v)
            dx = lax.shift_right_logical(kx, shift) & 255
            ix = p * CNT + dx * L + lane
            ox = plsc.load_gather(cntx, [ix])
            plsc.store_scatter(cntx, [ix], ox + 1)
            memx = rank_to_mem(ox)
            if store_keys:
                plsc.store_scatter(xk_dst, [memx], kx)
            val = (v * L + lane) if first else xv_src[pl.ds(v * L, L)]
            plsc.store_scatter(xv_dst, [memx], val)

        def permy(v, p):
            ky = loady(v)
            dy = lax.shift_right_logical(ky, shift) & 255
            iy = p * CNT + dy * L + lane
            oy = plsc.load_gather(cnty, [iy])
            plsc.store_scatter(cnty, [iy], oy + 1)
            memy = rank_to_mem(oy)
            if not last:
                plsc.store_scatter(yk_dst, [memy], ky)
            else:
                pos = plsc.load_gather(xv_dst, [memy])
                plsc.store_scatter(out, [pos], _decode(ky))

        if not last:
            def perm(i, _):
                for u in range(2):
                    for p in range(PART):
                        v = i * 2 + u + p * H
                        permx(v, p, True)
                        permy(v, p)
                return 0
            lax.fori_loop(0, H // 2, perm, 0)
        else:
            def perm_x(i, _):
                for u in range(2):
                    for p in range(PART):
                        permx(i * 2 + u + p * H, p, False)
                return 0
            lax.fori_loop(0, H // 2, perm_x, 0)

            def perm_y(i, _):
                for u in range(2):
                    for p in range(PART):
                        permy(i * 2 + u + p * H, p)
                return 0
            lax.fori_loop(0, H // 2, perm_y, 0)

    def column(j, _):
        col = wid * CPW + j
        pltpu.sync_copy(x_hbm.at[col], ak)
        pltpu.sync_copy(y_hbm.at[col], yb)
        radix_pass(ak, None, bv, yb, 0, True, False, xk_dst=bk, yk_dst=ck)
        radix_pass(bk, bv, av, ck, 8, False, False, xk_dst=ak, yk_dst=yb)
        radix_pass(ak, av, bv, yb, 16, False, False, xk_dst=bk, yk_dst=ck)
        radix_pass(bk, bv, av, ck, 24, False, True, out=ak)
        pltpu.sync_copy(ak, out_hbm.at[col])
        return 0

    lax.fori_loop(0, CPW, column, 0)


def _sc_transport(xT_bits, yT_bits):
    mesh = plsc.VectorSubcoreMesh(core_axis_name="c", subcore_axis_name="s",
                                  num_cores=NC, num_subcores=NS)
    f = pl.kernel(
        _sc_body,
        out_type=jax.ShapeDtypeStruct((C, N), jnp.int32),
        mesh=mesh,
        compiler_params=pltpu.CompilerParams(needs_layout_passes=False),
        scratch_types=[
            pltpu.VMEM((N,), jnp.int32),
            pltpu.VMEM((N,), jnp.int32),
            pltpu.VMEM((N,), jnp.int32),
            pltpu.VMEM((N,), jnp.int32),
            pltpu.VMEM((N,), jnp.int32),
            pltpu.VMEM((N,), jnp.int32),
            pltpu.VMEM((PART * CNT,), jnp.int32),
            pltpu.VMEM((PART * CNT,), jnp.int32),
        ],
    )
    return f(xT_bits, yT_bits)


def _assemble_body(scale_ref, t_ref, xp_ref, x_ref, th_ref, o_ref):
    th = _normalize_theta(th_ref[...])
    transported = lax.bitcast_convert_type(t_ref[0], jnp.float32)
    diff = transported - xp_ref[0]
    dn = (((0,), (0,)), ((), ()))
    t = lax.dot_general(diff, th, dn,
                        preferred_element_type=jnp.float32,
                        precision=lax.Precision.HIGHEST)
    o_ref[0] = x_ref[0] + t * scale_ref[0]


def _assemble(transT_bits, xT, x, theta_raw, n_projections):
    grid = (B, N // NT)
    scale = (1.0 / jnp.asarray(n_projections, jnp.float32)).reshape(1)
    return pl.pallas_call(
        _assemble_body,
        grid=grid,
        in_specs=[
            pl.BlockSpec(memory_space=pltpu.SMEM),
            pl.BlockSpec((1, P, NT), lambda b, n: (b, 0, n)),
            pl.BlockSpec((1, P, NT), lambda b, n: (b, 0, n)),
            pl.BlockSpec((1, NT, D), lambda b, n: (b, n, 0)),
            pl.BlockSpec((P, D), lambda b, n: (0, 0)),
        ],
        out_specs=pl.BlockSpec((1, NT, D), lambda b, n: (b, n, 0)),
        out_shape=jax.ShapeDtypeStruct((B, N, D), jnp.float32),
    )(scale, transT_bits, xT, x, theta_raw)


def kernel(x_batch, y_batch, eps, n_projections, theta_raw):
    del eps
    xT, yT = _project(x_batch, y_batch, theta_raw)
    xT_bits = lax.bitcast_convert_type(xT, jnp.int32).reshape(C, N)
    yT_bits = lax.bitcast_convert_type(yT, jnp.int32).reshape(C, N)
    transT_bits = _sc_transport(xT_bits, yT_bits).reshape(B, P, N)
    return _assemble(transT_bits, xT, x_batch, theta_raw, n_projections)
